# baseline (speedup 1.0000x reference)
import functools

import jax
import jax.numpy as jnp
from jax import lax
from jax.experimental import pallas as pl
from jax.experimental.pallas import tpu as pltpu
from jax.experimental.pallas import tpu_sc as plsc

SEQ_LEN = 8192
EMBED_DIM = 1024

_info = plsc.get_sparse_core_info()
_NC, _NS = _info.num_cores, _info.num_subcores
_NW = _NC * _NS
_ROWS_PER_W = SEQ_LEN // _NW

_mesh = plsc.VectorSubcoreMesh(core_axis_name="c", subcore_axis_name="s")


@functools.partial(
    pl.kernel,
    mesh=_mesh,
    out_type=jax.ShapeDtypeStruct((SEQ_LEN, EMBED_DIM), jnp.float32),
)
def _copy_rows(emb_hbm, out_hbm):
    wid = lax.axis_index("s") * _NC + lax.axis_index("c")
    base = wid * _ROWS_PER_W
    pltpu.sync_copy(
        emb_hbm.at[pl.ds(base, _ROWS_PER_W)],
        out_hbm.at[pl.ds(base, _ROWS_PER_W)],
    )


def kernel(x, embedding):
    del x
    return _copy_rows(embedding)


# speedup vs baseline: 24.3054x; 24.3054x over previous
import functools

import jax
import jax.numpy as jnp
from jax import lax
from jax.experimental import pallas as pl
from jax.experimental.pallas import tpu as pltpu
from jax.experimental.pallas import tpu_sc as plsc

SEQ_LEN = 8192
EMBED_DIM = 1024

_info = plsc.get_sparse_core_info()
_NC, _NS = _info.num_cores, _info.num_subcores
_NW = _NC * _NS
_ROWS_PER_W = SEQ_LEN // _NW
_CH = 32
_NCHUNK = _ROWS_PER_W // _CH

_mesh = plsc.VectorSubcoreMesh(core_axis_name="c", subcore_axis_name="s")


@functools.partial(
    pl.kernel,
    mesh=_mesh,
    out_type=jax.ShapeDtypeStruct((SEQ_LEN, EMBED_DIM), jnp.float32),
    scratch_types=[
        pltpu.VMEM_SHARED((_NS, 2, _CH, EMBED_DIM), jnp.float32),
        pltpu.SemaphoreType.DMA((2,)),
        pltpu.SemaphoreType.DMA((2,)),
    ],
)
def _copy_rows(emb_hbm, out_hbm, stage, in_sems, out_sems):
    c = lax.axis_index("c")
    s = lax.axis_index("s")
    wid = s * _NC + c
    base = wid * _ROWS_PER_W

    def in_copy(i):
        return pltpu.make_async_copy(
            emb_hbm.at[pl.ds(base + i * _CH, _CH)],
            stage.at[s, i % 2],
            in_sems.at[i % 2],
        )

    def out_copy(i):
        return pltpu.make_async_copy(
            stage.at[s, i % 2],
            out_hbm.at[pl.ds(base + i * _CH, _CH)],
            out_sems.at[i % 2],
        )

    in_copy(0).start()
    in_copy(1).start()
    in_copy(0).wait()
    out_copy(0).start()
    for i in range(1, _NCHUNK):
        in_copy(i).wait()
        out_copy(i).start()
        out_copy(i - 1).wait()
        if i + 1 < _NCHUNK:
            in_copy(i + 1).start()
    out_copy(_NCHUNK - 1).wait()


def kernel(x, embedding):
    del x
    return _copy_rows(embedding)
